# baseline (speedup 1.0000x reference)
import jax
import jax.numpy as jnp
from jax import lax
from jax.experimental import pallas as pl
from jax.experimental.pallas import tpu as pltpu
from jax.experimental.pallas import tpu_sc as plsc

B = 16
G = 100
P = 500
L = 16
STEPS = 32

_HALF_LOG_A = -0.0010010006671670687


def _sc_body(gt_hbm, idx_hbm, t_hbm, nz_hbm, out_hbm,
             gt_v, idx_v, t_v, nz_v, out_v, sem):
    b = lax.axis_index("s")

    cp_gt = pltpu.async_copy(gt_hbm.at[pl.ds(b * (G * 4), G * 4)], gt_v, sem)
    cp_ix = pltpu.async_copy(idx_hbm.at[b], idx_v.at[pl.ds(0, P)], sem)
    cp_t = pltpu.async_copy(t_hbm.at[b], t_v.at[pl.ds(0, P)], sem)
    cp_nz = pltpu.async_copy(nz_hbm.at[b], nz_v.at[pl.ds(0, P * 4)], sem)
    cp_gt.wait()
    cp_ix.wait()
    cp_t.wait()
    cp_nz.wait()

    lane4 = lax.iota(jnp.int32, 16) * 4

    def _step(i, carry):
        off = i * L
        g = jnp.minimum(jnp.maximum(idx_v[pl.ds(off, L)], 0), G - 1)
        tf = t_v[pl.ds(off, L)].astype(jnp.float32)
        sa = jnp.exp(tf * _HALF_LOG_A)
        x = 1.0 - sa * sa
        y = lax.bitcast_convert_type(
            0x5F3759DF - (lax.bitcast_convert_type(x, jnp.int32) >> 1),
            jnp.float32)
        for _ in range(2):
            y = y * (1.5 - 0.5 * x * y * y)
        sb = x * y * 1024.0
        gi = g * 4
        voff = off * 4
        for c in range(4):
            nidx = lane4 + (voff + c)
            gv = plsc.load_gather(gt_v, [gi + c])
            nv = plsc.load_gather(nz_v, [nidx])
            plsc.store_scatter(out_v, [nidx], gv * sa + nv * sb)
        return carry

    CH = (STEPS // 2) * L * 4
    lax.fori_loop(0, STEPS // 2, _step, 0)
    cp_a = pltpu.async_copy(out_v.at[pl.ds(0, CH)],
                            out_hbm.at[pl.ds(b * (P * 4), CH)], sem)
    lax.fori_loop(STEPS // 2, STEPS, _step, 0)
    pltpu.sync_copy(out_v.at[pl.ds(CH, P * 4 - CH)],
                    out_hbm.at[pl.ds(b * (P * 4) + CH, P * 4 - CH)])
    cp_a.wait()


@jax.jit
def kernel(gt_boxes, sampled_indices, t, noise):
    idx2 = sampled_indices.astype(jnp.int32)
    t2 = t.astype(jnp.int32)
    nz2 = noise.reshape(B, P * 4)
    gt_flat = gt_boxes.reshape(-1)

    sc = pl.kernel(
        _sc_body,
        out_type=jax.ShapeDtypeStruct((B * P * 4,), jnp.float32),
        mesh=plsc.VectorSubcoreMesh(core_axis_name="c", subcore_axis_name="s",
                                    num_cores=1),
        compiler_params=pltpu.CompilerParams(needs_layout_passes=False,
                                             use_tc_tiling_on_sc=False,
                                             disable_bounds_checks=True),
        scratch_types=[
            pltpu.VMEM((G * 4,), jnp.float32),
            pltpu.VMEM((512,), jnp.int32),
            pltpu.VMEM((512,), jnp.int32),
            pltpu.VMEM((2048,), jnp.float32),
            pltpu.VMEM((2048,), jnp.float32),
            pltpu.SemaphoreType.DMA,
        ],
    )
    out_flat = sc(gt_flat, idx2, t2, nz2)
    prior = out_flat.reshape(B, P, 4)
    return prior, t, sampled_indices


# speedup vs baseline: 1.0056x; 1.0056x over previous
import jax
import jax.numpy as jnp
from jax import lax
from jax.experimental import pallas as pl
from jax.experimental.pallas import tpu as pltpu
from jax.experimental.pallas import tpu_sc as plsc

B = 16
G = 100
P = 500
L = 16
STEPS = 32

_HALF_LOG_A = -0.0010010006671670687


def _sc_body(gt_hbm, idx_hbm, t_hbm, nz_hbm, out_hbm,
             gt_v, idx_v, t_v, nz_v, out_v, sem):
    b = lax.axis_index("s")

    cp_gt = pltpu.async_copy(gt_hbm.at[pl.ds(b * (G * 4), G * 4)], gt_v, sem)
    cp_ix = pltpu.async_copy(idx_hbm.at[b], idx_v.at[pl.ds(0, P)], sem)
    cp_t = pltpu.async_copy(t_hbm.at[b], t_v.at[pl.ds(0, P)], sem)
    cp_nz = pltpu.async_copy(nz_hbm.at[b], nz_v.at[pl.ds(0, P * 4)], sem)
    cp_gt.wait()
    cp_ix.wait()
    cp_t.wait()
    cp_nz.wait()

    lane4 = lax.iota(jnp.int32, 16) * 4

    def _step(i, carry):
        off = i * L
        g = jnp.minimum(jnp.maximum(idx_v[pl.ds(off, L)], 0), G - 1)
        tf = t_v[pl.ds(off, L)].astype(jnp.float32)
        sa = jnp.exp(tf * _HALF_LOG_A)
        x = 1.0 - sa * sa
        y = lax.bitcast_convert_type(
            0x5F3759DF - (lax.bitcast_convert_type(x, jnp.int32) >> 1),
            jnp.float32)
        for _ in range(2):
            y = y * (1.5 - 0.5 * x * y * y)
        sb = x * y * 1024.0
        gi = g * 4
        voff = off * 4
        for c in range(4):
            nidx = lane4 + (voff + c)
            gv = plsc.load_gather(gt_v, [gi + c])
            nv = plsc.load_gather(nz_v, [nidx])
            plsc.store_scatter(out_v, [nidx], gv * sa + nv * sb)
        return carry

    lax.fori_loop(0, STEPS, _step, 0)

    pltpu.sync_copy(out_v.at[pl.ds(0, P * 4)],
                    out_hbm.at[pl.ds(b * (P * 4), P * 4)])


@jax.jit
def kernel(gt_boxes, sampled_indices, t, noise):
    idx2 = sampled_indices.astype(jnp.int32)
    t2 = t.astype(jnp.int32)
    nz2 = noise.reshape(B, P * 4)
    gt_flat = gt_boxes.reshape(-1)

    sc = pl.kernel(
        _sc_body,
        out_type=jax.ShapeDtypeStruct((B * P * 4,), jnp.float32),
        mesh=plsc.VectorSubcoreMesh(core_axis_name="c", subcore_axis_name="s",
                                    num_cores=1),
        compiler_params=pltpu.CompilerParams(needs_layout_passes=False,
                                             use_tc_tiling_on_sc=False,
                                             disable_bounds_checks=True),
        scratch_types=[
            pltpu.VMEM((G * 4,), jnp.float32),
            pltpu.VMEM((512,), jnp.int32),
            pltpu.VMEM((512,), jnp.int32),
            pltpu.VMEM((2048,), jnp.float32),
            pltpu.VMEM((2048,), jnp.float32),
            pltpu.SemaphoreType.DMA,
        ],
    )
    out_flat = sc(gt_flat, idx2, t2, nz2)
    prior = out_flat.reshape(B, P, 4)
    return prior, t, sampled_indices
